# baseline (speedup 1.0000x reference)
import jax
import jax.numpy as jnp
from jax import lax
from jax.experimental import pallas as pl
from jax.experimental.pallas import tpu as pltpu
from jax.experimental.pallas import tpu_sc as plsc

_WORD_NUM = 1000000
_ENTITY_NUM = 1000000
_EMBED = 32
_FACTOR = 0.5
_L2 = 1e-06
_B = 16384
_L = 20
_K = 64

_NC, _NS = 2, 16
_NW = _NC * _NS
_BC = _B // _NW
_QC = _B * _L // _NW
_QCH = 5
_QROWS = _QC // _QCH
_IW = 64


def _sc_gather_body(ent_hbm, word_hbm, ebias_hbm, wbias_hbm,
                    users_hbm, items_hbm, rev_hbm, qw_hbm, negi_hbm, negw_hbm,
                    user_out, item_out, rev_out, q_out, ibias_out, rbias_out,
                    negi_out, negw_out, negib_out, negwb_out,
                    idx8, idxq, idx1, rows512, rowsq, bias512, sem):
    c = lax.axis_index("c")
    s = lax.axis_index("s")
    wid = s * _NC + c
    base = wid * _BC

    def gather512(tbl, dst):
        cps = [pltpu.async_copy(tbl.at[idx8.at[j]],
                                dst.at[pl.ds(j * _IW, _IW)], sem)
               for j in range(8)]
        for cp in cps:
            cp.wait()

    pltpu.sync_copy(users_hbm.at[pl.ds(wid * 8, 8)], idx8)
    gather512(ent_hbm, rows512)
    pltpu.sync_copy(rows512, user_out.at[pl.ds(base, _BC)])

    pltpu.sync_copy(items_hbm.at[pl.ds(wid * 8, 8)], idx8)
    gather512(ent_hbm, rows512)
    pltpu.sync_copy(rows512, item_out.at[pl.ds(base, _BC)])
    gather512(ebias_hbm, bias512)
    pltpu.sync_copy(bias512, ibias_out.at[pl.ds(base, _BC)])

    pltpu.sync_copy(rev_hbm.at[pl.ds(wid * 8, 8)], idx8)
    gather512(word_hbm, rows512)
    pltpu.sync_copy(rows512, rev_out.at[pl.ds(base, _BC)])
    gather512(wbias_hbm, bias512)
    pltpu.sync_copy(bias512, rbias_out.at[pl.ds(base, _BC)])

    for ch in range(_QCH):
        pltpu.sync_copy(qw_hbm.at[pl.ds(wid * (_QC // _IW) + ch * 32, 32)], idxq)
        cps = [pltpu.async_copy(word_hbm.at[idxq.at[j]],
                                rowsq.at[pl.ds(j * _IW, _IW)], sem)
               for j in range(32)]
        for cp in cps:
            cp.wait()
        pltpu.sync_copy(rowsq, q_out.at[pl.ds(wid * _QC + ch * _QROWS, _QROWS)])

    @pl.when(wid == 0)
    def _():
        pltpu.sync_copy(negi_hbm, idx1)
        pltpu.async_copy(ent_hbm.at[idx1.at[0]], rows512.at[pl.ds(0, _K)], sem).wait()
        pltpu.sync_copy(rows512.at[pl.ds(0, _K)], negi_out)
        pltpu.async_copy(ebias_hbm.at[idx1.at[0]], bias512.at[pl.ds(0, _K)], sem).wait()
        pltpu.sync_copy(bias512.at[pl.ds(0, _K)], negib_out)
        pltpu.sync_copy(negw_hbm, idx1)
        pltpu.async_copy(word_hbm.at[idx1.at[0]], rows512.at[pl.ds(0, _K)], sem).wait()
        pltpu.sync_copy(rows512.at[pl.ds(0, _K)], negw_out)
        pltpu.async_copy(wbias_hbm.at[idx1.at[0]], bias512.at[pl.ds(0, _K)], sem).wait()
        pltpu.sync_copy(bias512.at[pl.ds(0, _K)], negwb_out)


_sc_gather = pl.kernel(
    _sc_gather_body,
    out_type=[
        jax.ShapeDtypeStruct((_B, _EMBED), jnp.float32),
        jax.ShapeDtypeStruct((_B, _EMBED), jnp.float32),
        jax.ShapeDtypeStruct((_B, _EMBED), jnp.float32),
        jax.ShapeDtypeStruct((_B * _L, _EMBED), jnp.float32),
        jax.ShapeDtypeStruct((_B, 1), jnp.float32),
        jax.ShapeDtypeStruct((_B, 1), jnp.float32),
        jax.ShapeDtypeStruct((_K, _EMBED), jnp.float32),
        jax.ShapeDtypeStruct((_K, _EMBED), jnp.float32),
        jax.ShapeDtypeStruct((_K, 1), jnp.float32),
        jax.ShapeDtypeStruct((_K, 1), jnp.float32),
    ],
    mesh=plsc.VectorSubcoreMesh(core_axis_name="c", subcore_axis_name="s",
                                num_cores=_NC, num_subcores=_NS),
    compiler_params=pltpu.CompilerParams(use_tc_tiling_on_sc=False),
    scratch_types=[
        pltpu.VMEM((8, _IW), jnp.int32),
        pltpu.VMEM((32, _IW), jnp.int32),
        pltpu.VMEM((1, _K), jnp.int32),
        pltpu.VMEM((_BC, _EMBED), jnp.float32),
        pltpu.VMEM((_QROWS, _EMBED), jnp.float32),
        pltpu.VMEM((_BC, 1), jnp.float32),
        pltpu.SemaphoreType.DMA,
    ],
)


_RN = 8000
_GN = _WORD_NUM // _RN


def _norm_body(w_ref, e_ref, o_ref, acc):
    i = pl.program_id(0)

    @pl.when(i == 0)
    def _():
        acc[0] = 0.0
        acc[1] = 0.0

    w = w_ref[...]
    e = e_ref[...]
    acc[0] += jnp.sum(w * w)
    acc[1] += jnp.sum(e * e)

    @pl.when(i == _GN - 1)
    def _():
        o_ref[0, 0] = jnp.sqrt(acc[0])
        o_ref[0, 1] = jnp.sqrt(acc[1])


def _norms(word_embedding, entity_embedding):
    return pl.pallas_call(
        _norm_body,
        grid=(_GN,),
        in_specs=[pl.BlockSpec((_RN, _EMBED), lambda i: (i, 0)),
                  pl.BlockSpec((_RN, _EMBED), lambda i: (i, 0))],
        out_specs=pl.BlockSpec(memory_space=pltpu.SMEM),
        out_shape=jax.ShapeDtypeStruct((1, 2), jnp.float32),
        scratch_shapes=[pltpu.SMEM((2,), jnp.float32)],
    )(word_embedding, entity_embedding)


_GB = 16
_BCH = _B // _GB


def _softplus(x):
    return jnp.maximum(x, 0.0) + jnp.log1p(jnp.exp(-jnp.abs(x)))


def _nce_body(q3, user, item, rev, ib, rb, wq, bq2, negi, negw, nib, nwb, nrm,
              o_ref, acc):
    i = pl.program_id(0)

    @pl.when(i == 0)
    def _():
        acc[0] = 0.0

    q = jnp.sum(q3[...], axis=1) * (1.0 / _L)
    qp = jnp.tanh(
        lax.dot_general(q, wq[...], (((1,), (1,)), ((), ())),
                        preferred_element_type=jnp.float32) + bq2[...])
    u = user[...]
    it = item[...]
    rv = rev[...]
    pers = _FACTOR * qp + (1.0 - _FACTOR) * u

    ngw = negw[...]
    ngi = negi[...]

    def nll(anchor, pos, pb, negs, nb):
        pos_s = jnp.sum(anchor * pos, axis=1, keepdims=True) + pb
        neg_s = lax.dot_general(anchor, negs, (((1,), (1,)), ((), ())),
                                preferred_element_type=jnp.float32) + nb
        return jnp.sum(_softplus(-pos_s)) + jnp.sum(_softplus(neg_s))

    total = (nll(u, rv, rb[...], ngw, nwb[...])
             + nll(it, rv, rb[...], ngw, nwb[...])
             + nll(pers, it, ib[...], ngi, nib[...]))
    acc[0] += total

    @pl.when(i == _GB - 1)
    def _():
        o_ref[0, 0] = acc[0] * (1.0 / _B) + _L2 * (nrm[0, 0] + nrm[0, 1])


def _nce(q_rows, user_rows, item_rows, rev_rows, ibias, rbias,
         Wq, bq2, negi_rows, negw_rows, nib, nwb, nrm):
    return pl.pallas_call(
        _nce_body,
        grid=(_GB,),
        in_specs=[
            pl.BlockSpec((_BCH, _L, _EMBED), lambda i: (i, 0, 0)),
            pl.BlockSpec((_BCH, _EMBED), lambda i: (i, 0)),
            pl.BlockSpec((_BCH, _EMBED), lambda i: (i, 0)),
            pl.BlockSpec((_BCH, _EMBED), lambda i: (i, 0)),
            pl.BlockSpec((_BCH, 1), lambda i: (i, 0)),
            pl.BlockSpec((_BCH, 1), lambda i: (i, 0)),
            pl.BlockSpec((_EMBED, _EMBED), lambda i: (0, 0)),
            pl.BlockSpec((1, _EMBED), lambda i: (0, 0)),
            pl.BlockSpec((_K, _EMBED), lambda i: (0, 0)),
            pl.BlockSpec((_K, _EMBED), lambda i: (0, 0)),
            pl.BlockSpec((1, _K), lambda i: (0, 0)),
            pl.BlockSpec((1, _K), lambda i: (0, 0)),
            pl.BlockSpec(memory_space=pltpu.SMEM),
        ],
        out_specs=pl.BlockSpec(memory_space=pltpu.SMEM),
        out_shape=jax.ShapeDtypeStruct((1, 1), jnp.float32),
        scratch_shapes=[pltpu.SMEM((1,), jnp.float32)],
    )(q_rows.reshape(_B, _L, _EMBED), user_rows, item_rows, rev_rows,
      ibias, rbias, Wq, bq2, negi_rows, negw_rows, nib, nwb, nrm)


def kernel(word_embedding, word_bias, entity_embedding, entity_bias, Wq, bq,
           users, items, query_words, review_words, neg_items, neg_review_words):
    i32 = lambda x: x.astype(jnp.int32)
    users2 = i32(users).reshape(-1, _IW)
    items2 = i32(items).reshape(-1, _IW)
    rev2 = i32(review_words).reshape(-1, _IW)
    qw2 = i32(query_words).reshape(-1, _IW)
    negi2 = i32(neg_items).reshape(1, _K)
    negw2 = i32(neg_review_words).reshape(1, _K)

    (user_rows, item_rows, rev_rows, q_rows, ibias, rbias,
     negi_rows, negw_rows, negib, negwb) = _sc_gather(
        entity_embedding, word_embedding, entity_bias, word_bias,
        users2, items2, rev2, qw2, negi2, negw2)

    nrm = _norms(word_embedding, entity_embedding)

    loss = _nce(q_rows, user_rows, item_rows, rev_rows, ibias, rbias,
                Wq, bq.reshape(1, _EMBED), negi_rows, negw_rows,
                negib.reshape(1, _K), negwb.reshape(1, _K), nrm)
    return loss.reshape(())
